# baseline (speedup 1.0000x reference)
import functools

import jax
import jax.numpy as jnp
from jax.experimental import pallas as pl
from jax.experimental.pallas import tpu as pltpu

N = 100000
D_IN, D_HID, D_ATT = 1024, 512, 256
N_CLASSES = 2
BN = 2000
GRID = N // BN


def _block_kernel(x_ref, w1_ref, b1_ref, wau_ref, bau_ref,
                  ww_ref, bw_ref, m_ref, z_ref, macc_ref, h_s, au_s):
    i = pl.program_id(0)
    slot = jax.lax.rem(i, 2)

    @pl.when(i < GRID)
    def _head():
        h = jnp.maximum(
            jnp.dot(x_ref[...], w1_ref[...],
                    preferred_element_type=jnp.float32)
            + b1_ref[...], 0.0)
        au = jnp.dot(h, wau_ref[...],
                     preferred_element_type=jnp.float32) + bau_ref[...]
        h_s[pl.ds(slot, 1)] = h[None]
        au_s[pl.ds(slot, 1)] = au[None]

    @pl.when(i > 0)
    def _tail():
        prev = 1 - slot
        h = h_s[pl.ds(prev, 1)][0]
        au = au_s[pl.ds(prev, 1)][0]
        ag = jnp.tanh(au[:, :D_ATT]) * jax.nn.sigmoid(au[:, D_ATT:])
        s = jnp.sum(ag * ww_ref[...], axis=1, keepdims=True) + bw_ref[...]
        m_i = jnp.max(s, axis=0, keepdims=True)
        p = jnp.exp(s - m_i)
        m_ref[...] = m_i.reshape(1, 1, 1)
        z_ref[...] = jnp.sum(p, axis=0, keepdims=True).reshape(1, 1, 1)
        macc_ref[...] = jax.lax.dot_general(
            p, h, (((0,), (0,)), ((), ())),
            preferred_element_type=jnp.float32).reshape(1, 1, D_HID)


def _combine_kernel(m_ref, z_ref, macc_ref, wc_ref, bc_ref,
                    logits_ref, yhat_ref, yprob_ref):
    m = m_ref[...]
    m_star = jnp.max(m, axis=0, keepdims=True)
    w = jnp.exp(m - m_star)
    z = jnp.sum(w * z_ref[...], axis=0, keepdims=True)
    M = jnp.sum(w * macc_ref[...], axis=0, keepdims=True) / z
    logits = jnp.dot(M, wc_ref[...],
                     preferred_element_type=jnp.float32) + bc_ref[...]
    logits_ref[...] = logits
    e = jnp.exp(logits - jnp.max(logits, axis=1, keepdims=True))
    yprob_ref[...] = e / jnp.sum(e, axis=1, keepdims=True)
    yhat_ref[...] = (logits[:, 1:2] > logits[:, 0:1]).astype(jnp.int32)


@functools.partial(jax.jit, static_argnames=("interpret",))
def kernel(x, W1, b1, Wa, ba, Wu, bu, Ww, bw, Wc, bc, interpret=False):
    full = lambda shape: pl.BlockSpec(shape, lambda i: (0, 0))
    m, z, macc = pl.pallas_call(
        _block_kernel,
        grid=(GRID + 1,),
        in_specs=[
            pl.BlockSpec((BN, D_IN),
                         lambda i: (jnp.minimum(i, GRID - 1), 0)),
            pl.BlockSpec((D_IN, D_HID), lambda i: (0, 0)),
            full((1, D_HID)),
            full((D_HID, 2 * D_ATT)),
            full((1, 2 * D_ATT)),
            full((1, D_ATT)),
            full((1, 1)),
        ],
        out_specs=[
            pl.BlockSpec((1, 1, 1), lambda i: (jnp.maximum(i - 1, 0), 0, 0)),
            pl.BlockSpec((1, 1, 1), lambda i: (jnp.maximum(i - 1, 0), 0, 0)),
            pl.BlockSpec((1, 1, D_HID),
                         lambda i: (jnp.maximum(i - 1, 0), 0, 0)),
        ],
        out_shape=[
            jax.ShapeDtypeStruct((GRID, 1, 1), jnp.float32),
            jax.ShapeDtypeStruct((GRID, 1, 1), jnp.float32),
            jax.ShapeDtypeStruct((GRID, 1, D_HID), jnp.float32),
        ],
        scratch_shapes=[
            pltpu.VMEM((2, BN, D_HID), jnp.float32),
            pltpu.VMEM((2, BN, 2 * D_ATT), jnp.float32),
        ],
        interpret=interpret,
    )(
        x, W1, b1.reshape(1, D_HID),
        jnp.concatenate([Wa, Wu], axis=1),
        jnp.concatenate([ba, bu]).reshape(1, 2 * D_ATT),
        Ww.reshape(1, D_ATT), bw.reshape(1, 1),
    )
    logits, yhat, yprob = pl.pallas_call(
        _combine_kernel,
        out_shape=[
            jax.ShapeDtypeStruct((1, N_CLASSES), jnp.float32),
            jax.ShapeDtypeStruct((1, 1), jnp.int32),
            jax.ShapeDtypeStruct((1, N_CLASSES), jnp.float32),
        ],
        interpret=interpret,
    )(m.reshape(GRID, 1), z.reshape(GRID, 1), macc.reshape(GRID, D_HID),
      Wc, bc.reshape(1, N_CLASSES))
    return logits, yhat.reshape((1,)), yprob


# speedup vs baseline: 1.1349x; 1.1349x over previous
import functools

import jax
import jax.numpy as jnp
from jax.experimental import pallas as pl
from jax.experimental.pallas import tpu as pltpu

N = 100000
D_IN, D_HID, D_ATT = 1024, 512, 256
N_CLASSES = 2
BN = 2000
GRID = N // BN


def _clam_kernel(x_ref, w1_ref, b1_ref, wa_ref, ba_ref, wu_ref, bu_ref,
                 ww_ref, bw_ref, wc_ref, bc_ref,
                 logits_ref, yhat_ref, yprob_ref,
                 m_s, z_s, macc_s):
    i = pl.program_id(0)

    @pl.when(i == 0)
    def _init():
        m_s[...] = jnp.full_like(m_s, -jnp.inf)
        z_s[...] = jnp.zeros_like(z_s)
        macc_s[...] = jnp.zeros_like(macc_s)

    x_blk = x_ref[...]
    h = jnp.maximum(
        jnp.dot(x_blk, w1_ref[...], preferred_element_type=jnp.float32)
        + b1_ref[...], 0.0)
    a = jnp.tanh(
        jnp.dot(h, wa_ref[...], preferred_element_type=jnp.float32)
        + ba_ref[...])
    g = jax.nn.sigmoid(
        jnp.dot(h, wu_ref[...], preferred_element_type=jnp.float32)
        + bu_ref[...])
    s = jnp.sum(a * g * ww_ref[...], axis=1, keepdims=True) + bw_ref[...]

    m_old = m_s[...]
    m_new = jnp.maximum(m_old, jnp.max(s, axis=0, keepdims=True))
    alpha = jnp.exp(m_old - m_new)
    p = jnp.exp(s - m_new)
    z_s[...] = z_s[...] * alpha + jnp.sum(p, axis=0, keepdims=True)
    macc_s[...] = macc_s[...] * alpha + jax.lax.dot_general(
        p, h, (((0,), (0,)), ((), ())),
        preferred_element_type=jnp.float32)
    m_s[...] = m_new

    @pl.when(i == GRID - 1)
    def _epilogue():
        M = macc_s[...] / z_s[...]
        logits = jnp.dot(M, wc_ref[...],
                         preferred_element_type=jnp.float32) + bc_ref[...]
        logits_ref[...] = logits
        e = jnp.exp(logits - jnp.max(logits, axis=1, keepdims=True))
        yprob_ref[...] = e / jnp.sum(e, axis=1, keepdims=True)
        yhat_ref[...] = (logits[:, 1:2] > logits[:, 0:1]).astype(jnp.int32)


@functools.partial(jax.jit, static_argnames=("interpret",))
def kernel(x, W1, b1, Wa, ba, Wu, bu, Ww, bw, Wc, bc, interpret=False):
    full = lambda shape: pl.BlockSpec(shape, lambda i: (0, 0))
    logits, yhat, yprob = pl.pallas_call(
        _clam_kernel,
        grid=(GRID,),
        in_specs=[
            pl.BlockSpec((BN, D_IN), lambda i: (i, 0)),
            full((D_IN, D_HID)),
            full((1, D_HID)),
            full((D_HID, D_ATT)),
            full((1, D_ATT)),
            full((D_HID, D_ATT)),
            full((1, D_ATT)),
            full((1, D_ATT)),
            full((1, 1)),
            full((D_HID, N_CLASSES)),
            full((1, N_CLASSES)),
        ],
        out_specs=[
            full((1, N_CLASSES)),
            full((1, 1)),
            full((1, N_CLASSES)),
        ],
        out_shape=[
            jax.ShapeDtypeStruct((1, N_CLASSES), jnp.float32),
            jax.ShapeDtypeStruct((1, 1), jnp.int32),
            jax.ShapeDtypeStruct((1, N_CLASSES), jnp.float32),
        ],
        scratch_shapes=[
            pltpu.VMEM((1, 1), jnp.float32),
            pltpu.VMEM((1, 1), jnp.float32),
            pltpu.VMEM((1, D_HID), jnp.float32),
        ],
        interpret=interpret,
    )(
        x, W1, b1.reshape(1, D_HID), Wa, ba.reshape(1, D_ATT),
        Wu, bu.reshape(1, D_ATT), Ww.reshape(1, D_ATT), bw.reshape(1, 1),
        Wc, bc.reshape(1, N_CLASSES),
    )
    return logits, yhat.reshape((1,)), yprob
